# baseline (speedup 1.0000x reference)
import functools

import jax
import jax.numpy as jnp
from jax import lax
from jax.experimental import pallas as pl
from jax.experimental.pallas import tpu as pltpu
from jax.experimental.pallas import tpu_sc as plsc

K = 8
B, S, C = 4, 8192, 1024
L = 16
NC, NS = 2, 16
NW = NC * NS
CPW = (B * C) // NW
NCHUNK = CPW // L
WIN = 128
NWIN = S // WIN
G = 16
GPW = WIN // G
NG = S // G
IDW = NG + L

NEG_INF = float("-inf")

SORT8 = [
    (0, 1), (2, 3), (4, 5), (6, 7),
    (0, 2), (1, 3), (4, 6), (5, 7),
    (1, 2), (5, 6),
    (0, 4), (1, 5), (2, 6), (3, 7),
    (2, 4), (3, 5),
    (1, 2), (3, 4), (5, 6),
]
BITONIC8 = [
    (0, 4), (1, 5), (2, 6), (3, 7),
    (0, 2), (1, 3), (4, 6), (5, 7),
    (0, 1), (2, 3), (4, 5), (6, 7),
]


def _apply_net(v, net):
    v = list(v)
    for a, b in net:
        hi = jnp.maximum(v[a], v[b])
        lo = jnp.minimum(v[a], v[b])
        v[a], v[b] = hi, lo
    return v


def _merge_top8(r, c):
    z = [jnp.maximum(r[i], c[K - 1 - i]) for i in range(K)]
    return _apply_net(z, BITONIC8)


def _tree_max(c):
    while len(c) > 1:
        c = [jnp.maximum(c[2 * i], c[2 * i + 1]) for i in range(len(c) // 2)]
    return c[0]


def kernel(x):
    mesh = plsc.VectorSubcoreMesh(core_axis_name="c", subcore_axis_name="s")

    @functools.partial(
        pl.kernel,
        out_type=jax.ShapeDtypeStruct((B, C * K), jnp.float32),
        mesh=mesh,
        scratch_types=[
            pltpu.VMEM((WIN, CPW), jnp.float32),
            pltpu.VMEM((WIN, CPW), jnp.float32),
            pltpu.VMEM((NG, CPW), jnp.float32),
            pltpu.VMEM((NCHUNK * IDW,), jnp.int32),
            pltpu.VMEM((K, CPW), jnp.float32),
            pltpu.VMEM((K, CPW), jnp.float32),
            pltpu.VMEM((K * CPW,), jnp.float32),
            pltpu.SMEM((NWIN * NCHUNK,), jnp.int32),
            pltpu.SMEM((NCHUNK,), jnp.int32),
            pltpu.SemaphoreType.DMA,
            pltpu.SemaphoreType.DMA,
        ],
        compiler_params=pltpu.CompilerParams(needs_layout_passes=False),
    )
    def run(x_hbm, out_hbm, buf0, buf1, mbuf, idl, gm, rbuf, obuf,
            counts_s, cur_s, sem0, sem1):
        wid = lax.axis_index("s") * NC + lax.axis_index("c")
        b = wid // (C // CPW)
        c0 = (wid % (C // CPW)) * CPW

        def src(w):
            return x_hbm.at[b, pl.ds(w * WIN, WIN), pl.ds(c0, CPW)]

        for j in range(NCHUNK):
            for k in range(K):
                gm[k, pl.ds(j * L, L)] = jnp.full((L,), NEG_INF)
                rbuf[k, pl.ds(j * L, L)] = jnp.full((L,), NEG_INF)

        def p1_window(buf, w):
            def jbody(j, _):
                ms = []
                for g in range(GPW):
                    c = [buf[g * G + t, pl.ds(j * L, L)] for t in range(G)]
                    m = _tree_max(c)
                    mbuf[w * GPW + g, pl.ds(j * L, L)] = m
                    ms.append(m)
                ms = _apply_net(ms, SORT8)
                gv = [gm[k, pl.ds(j * L, L)] for k in range(K)]
                gv = _merge_top8(gv, ms)
                for k in range(K):
                    gm[k, pl.ds(j * L, L)] = gv[k]
                return 0

            lax.fori_loop(0, NCHUNK, jbody, 0)

        pltpu.async_copy(src(0), buf0, sem0)

        @pl.loop(0, NWIN // 2)
        def _p1(p):
            w0 = 2 * p
            pltpu.async_copy(src(w0 + 1), buf1, sem1)
            pltpu.make_async_copy(src(0), buf0, sem0).wait()
            p1_window(buf0, w0)

            @pl.when(w0 + 2 < NWIN)
            def _():
                pltpu.async_copy(src(w0 + 2), buf0, sem0)

            pltpu.make_async_copy(src(0), buf1, sem1).wait()
            p1_window(buf1, w0 + 1)

        lane = lax.iota(jnp.int32, L)

        def jbody15(j, _):
            tau = gm[K - 1, pl.ds(j * L, L)]

            def bbody(bb, cur):
                bits = jnp.zeros((L,), jnp.int32)
                for r in range(L):
                    m = mbuf[bb * L + r, pl.ds(j * L, L)]
                    bits = bits | jnp.where(m >= tau, jnp.int32(1 << r), 0)
                for sh in (8, 4, 2, 1):
                    bits = bits | jnp.take(bits, (lane + sh) % L)
                mask = ((bits >> lane) & 1) == 1
                ids = bb * L + lane
                plsc.store_compressed(idl.at[pl.ds(j * IDW + cur, L)], ids,
                                      mask=mask)
                cnt_all = plsc.all_reduce_population_count(mask)[0]
                cnt_lo = plsc.all_reduce_population_count(
                    mask & (lane < GPW))[0]
                counts_s[(2 * bb) * NCHUNK + j] = cnt_lo
                counts_s[(2 * bb + 1) * NCHUNK + j] = cnt_all - cnt_lo
                return cur + cnt_all

            lax.fori_loop(0, NG // L, bbody, jnp.int32(0))
            cur_s[j] = 0
            return 0

        lax.fori_loop(0, NCHUNK, jbody15, 0)

        def p2_window(buf, w):
            def jbody2(j, _):
                n = counts_s[w * NCHUNK + j]
                cur = cur_s[j]
                r = tuple(rbuf[k, pl.ds(j * L, L)] for k in range(K))

                def gbody(i, r):
                    pidx = cur + i
                    vec = idl[pl.ds(j * IDW + (pidx // L) * L, L)]
                    gsp = jnp.take(vec, jnp.broadcast_to(pidx % L, (L,)))
                    base = (gsp[0] - w * GPW) * G
                    c = [buf[base + t, pl.ds(j * L, L)] for t in range(G)]
                    ca = _apply_net(c[:K], SORT8)
                    cb = _apply_net(c[K:], SORT8)
                    cm = _merge_top8(ca, cb)
                    return tuple(_merge_top8(list(r), cm))

                r = lax.fori_loop(0, n, gbody, r)
                for k in range(K):
                    rbuf[k, pl.ds(j * L, L)] = r[k]
                cur_s[j] = cur + n
                return 0

            lax.fori_loop(0, NCHUNK, jbody2, 0)

        pltpu.async_copy(src(0), buf0, sem0)

        @pl.loop(0, NWIN // 2)
        def _p2(p):
            w0 = 2 * p
            pltpu.async_copy(src(w0 + 1), buf1, sem1)
            pltpu.make_async_copy(src(0), buf0, sem0).wait()
            p2_window(buf0, w0)

            @pl.when(w0 + 2 < NWIN)
            def _():
                pltpu.async_copy(src(w0 + 2), buf0, sem0)

            pltpu.make_async_copy(src(0), buf1, sem1).wait()
            p2_window(buf1, w0 + 1)

        kmask = [(lane & (K - 1)) == k for k in range(K)]
        for t in range(CPW * K // L):
            ch0 = 2 * t
            j = ch0 // L
            m = ch0 % L
            idx = jnp.where(lane < K, m, m + 1)
            out = jnp.full((L,), NEG_INF)
            for k in range(K):
                g = jnp.take(rbuf[k, pl.ds(j * L, L)], idx)
                out = jnp.where(kmask[k], g, out)
            obuf[pl.ds(t * L, L)] = out
        pltpu.sync_copy(obuf, out_hbm.at[b, pl.ds(c0 * K, CPW * K)])

    return run(x)


# speedup vs baseline: 1.2434x; 1.2434x over previous
import functools

import jax
import jax.numpy as jnp
from jax import lax
from jax.experimental import pallas as pl
from jax.experimental.pallas import tpu as pltpu
from jax.experimental.pallas import tpu_sc as plsc

K = 8
B, S, C = 4, 8192, 1024
L = 16
NC, NS = 2, 16
NW = NC * NS
CPW = (B * C) // NW
NCHUNK = CPW // L
WIN = 256
NWIN = S // WIN

NEG_INF = float("-inf")

SORT8 = [
    (0, 1), (2, 3), (4, 5), (6, 7),
    (0, 2), (1, 3), (4, 6), (5, 7),
    (1, 2), (5, 6),
    (0, 4), (1, 5), (2, 6), (3, 7),
    (2, 4), (3, 5),
    (1, 2), (3, 4), (5, 6),
]
BITONIC8 = [
    (0, 4), (1, 5), (2, 6), (3, 7),
    (0, 2), (1, 3), (4, 6), (5, 7),
    (0, 1), (2, 3), (4, 5), (6, 7),
]


def _apply_net(v, net):
    v = list(v)
    for a, b in net:
        hi = jnp.maximum(v[a], v[b])
        lo = jnp.minimum(v[a], v[b])
        v[a], v[b] = hi, lo
    return v


def _merge_top8(r, c):
    z = [jnp.maximum(r[i], c[K - 1 - i]) for i in range(K)]
    return _apply_net(z, BITONIC8)


def _process_window(buf, rbuf):
    for j in range(NCHUNK):
        r = tuple(rbuf[k, pl.ds(j * L, L)] for k in range(K))

        def body(i, r, j=j):
            c = [buf[i * K + t, pl.ds(j * L, L)] for t in range(K)]
            c = _apply_net(c, SORT8)
            return tuple(_merge_top8(list(r), c))

        r = lax.fori_loop(0, WIN // K, body, r, unroll=2)
        for k in range(K):
            rbuf[k, pl.ds(j * L, L)] = r[k]


def kernel(x):
    mesh = plsc.VectorSubcoreMesh(core_axis_name="c", subcore_axis_name="s")

    @functools.partial(
        pl.kernel,
        out_type=jax.ShapeDtypeStruct((B, C * K), jnp.float32),
        mesh=mesh,
        scratch_types=[
            pltpu.VMEM((WIN, CPW), jnp.float32),
            pltpu.VMEM((WIN, CPW), jnp.float32),
            pltpu.VMEM((K, CPW), jnp.float32),
            pltpu.VMEM((K * CPW,), jnp.float32),
            pltpu.SemaphoreType.DMA,
            pltpu.SemaphoreType.DMA,
        ],
        compiler_params=pltpu.CompilerParams(needs_layout_passes=False),
    )
    def run(x_hbm, out_hbm, buf0, buf1, rbuf, obuf, sem0, sem1):
        wid = lax.axis_index("s") * NC + lax.axis_index("c")
        b = wid // (C // CPW)
        c0 = (wid % (C // CPW)) * CPW

        def src(w):
            return x_hbm.at[b, pl.ds(w * WIN, WIN), pl.ds(c0, CPW)]

        for j in range(NCHUNK):
            for k in range(K):
                rbuf[k, pl.ds(j * L, L)] = jnp.full((L,), NEG_INF)

        pltpu.async_copy(src(0), buf0, sem0)

        @pl.loop(0, NWIN // 2)
        def _pair(p):
            w0 = 2 * p
            pltpu.async_copy(src(w0 + 1), buf1, sem1)
            pltpu.make_async_copy(src(0), buf0, sem0).wait()
            _process_window(buf0, rbuf)

            @pl.when(w0 + 2 < NWIN)
            def _():
                pltpu.async_copy(src(w0 + 2), buf0, sem0)

            pltpu.make_async_copy(src(0), buf1, sem1).wait()
            _process_window(buf1, rbuf)

        lane = lax.iota(jnp.int32, L)
        kmask = [(lane & (K - 1)) == k for k in range(K)]
        for t in range(CPW * K // L):
            ch0 = 2 * t
            j = ch0 // L
            m = ch0 % L
            idx = jnp.where(lane < K, m, m + 1)
            out = jnp.full((L,), NEG_INF)
            for k in range(K):
                g = jnp.take(rbuf[k, pl.ds(j * L, L)], idx)
                out = jnp.where(kmask[k], g, out)
            obuf[pl.ds(t * L, L)] = out
        pltpu.sync_copy(obuf, out_hbm.at[b, pl.ds(c0 * K, CPW * K)])

    return run(x)
